# baseline (speedup 1.0000x reference)
import functools

import jax
import jax.numpy as jnp
from jax.experimental import pallas as pl
from jax.experimental.pallas import tpu as pltpu
from jax.experimental.pallas import tpu_sc as plsc

TOPK = 20
BLK = 8192


def _mm_kernel(q_ref, p_ref, s_ref):
    q = q_ref[...]
    p = p_ref[...]
    s_ref[...] = jax.lax.dot_general(
        q, p, (((1,), (1,)), ((), ())),
        preferred_element_type=jnp.float32,
    )


def _scores(q, p, blk):
    qn, d = q.shape
    k_total = p.shape[0]
    nb = pl.cdiv(k_total, blk)
    return pl.pallas_call(
        _mm_kernel,
        grid=(nb,),
        in_specs=[
            pl.BlockSpec((qn, d), lambda i: (0, 0)),
            pl.BlockSpec((blk, d), lambda i: (i, 0)),
        ],
        out_specs=pl.BlockSpec((qn, blk), lambda i: (0, i)),
        out_shape=jax.ShapeDtypeStruct((qn, nb * blk), jnp.float32),
        compiler_params=pltpu.CompilerParams(
            dimension_semantics=("arbitrary",),
        ),
    )(q, p)


def _sc_rowmax(scores_flat, row_stride, half, ch):
    mesh = plsc.VectorSubcoreMesh(core_axis_name="c", subcore_axis_name="s")

    @functools.partial(
        pl.kernel, mesh=mesh,
        out_type=jax.ShapeDtypeStruct((32, 16), jnp.float32),
        scratch_types=[
            pltpu.VMEM((ch,), jnp.float32),
            pltpu.VMEM((16,), jnp.float32),
        ],
    )
    def k(scores_hbm, out_hbm, buf, ovec):
        cid = jax.lax.axis_index("c")
        sid = jax.lax.axis_index("s")
        wid = sid * 2 + cid
        q = wid // 2
        h = wid % 2
        base = q * row_stride + h * half

        def chunk_body(ci, acc):
            pltpu.sync_copy(scores_hbm.at[pl.ds(base + ci * ch, ch)], buf)

            def vbody(j, a):
                return jnp.maximum(a, buf[pl.ds(j * 16, 16)])

            return jax.lax.fori_loop(0, ch // 16, vbody, acc)

        acc = jax.lax.fori_loop(
            0, half // ch, chunk_body,
            jnp.full((16,), -jnp.inf, jnp.float32))
        ovec[...] = acc
        pltpu.sync_copy(ovec, out_hbm.at[wid])

    return k(scores_flat)


def kernel(question_embeddings, passage_embeddings, topk):
    del topk
    q = question_embeddings
    scores = _scores(q, passage_embeddings, BLK)
    rmax = _sc_rowmax(scores.reshape(-1), scores.shape[1], 50000, 10000)
    qn = q.shape[0]
    logits = jnp.broadcast_to(jnp.max(rmax), (qn, TOPK))
    idx = jnp.zeros((qn, TOPK), jnp.int32) + jnp.max(rmax).astype(jnp.int32)
    return logits, idx, q


# speedup vs baseline: 1.3799x; 1.3799x over previous
import functools

import jax
import jax.numpy as jnp
from jax.experimental import pallas as pl
from jax.experimental.pallas import tpu as pltpu

TOPK = 20
BLK = 9216
PAD = 128


def _topk_kernel(q_ref, p_ref, vals_ref, idx_ref, rv, ri, sc_ref, *,
                 k_total, blk):
    i = pl.program_id(0)
    nb = pl.num_programs(0)
    qn = q_ref.shape[0]

    @pl.when(i == 0)
    def _init():
        rv[...] = jnp.full((qn, PAD), -jnp.inf, jnp.float32)
        ri[...] = jnp.zeros((qn, PAD), jnp.int32)

    q = q_ref[...]
    p = p_ref[...]
    scores = jax.lax.dot_general(
        q, p, (((1,), (1,)), ((), ())),
        preferred_element_type=jnp.float32,
    )
    lane = jax.lax.broadcasted_iota(jnp.int32, (qn, blk), 1)
    rem = k_total - i * blk
    scores = jnp.where(lane < rem, scores, -jnp.inf)
    l20 = jax.lax.broadcasted_iota(jnp.int32, (qn, PAD), 1)

    def th_of(rv_v):
        return jnp.max(jnp.where(l20 == TOPK - 1, rv_v, -jnp.inf),
                       axis=1, keepdims=True)

    def cond(c):
        rv_v, _ri_v, m = c
        return jnp.any(m > th_of(rv_v))

    def body(c):
        rv_v, ri_v, m = c
        sc = sc_ref[...]
        take = m > th_of(rv_v)
        sel = jnp.min(jnp.where(sc == m, lane, blk), axis=1, keepdims=True)
        hit = lane == sel
        idx_t = i * blk + sel
        pos = jnp.sum(jnp.where(rv_v >= m, 1, 0), axis=1, keepdims=True)
        sv = jnp.roll(rv_v, 1, axis=1)
        si = jnp.roll(ri_v, 1, axis=1)
        nrv = jnp.where(l20 < pos, rv_v, jnp.where(l20 == pos, m, sv))
        nri = jnp.where(l20 < pos, ri_v, jnp.where(l20 == pos, idx_t, si))
        rv2 = jnp.where(take, nrv, rv_v)
        ri2 = jnp.where(take, nri, ri_v)
        sc2 = jnp.where(hit & take, -jnp.inf, sc)
        sc_ref[...] = sc2
        m2 = jnp.max(sc2, axis=1, keepdims=True)
        return rv2, ri2, m2

    sc_ref[...] = scores
    m0 = jnp.max(scores, axis=1, keepdims=True)
    nv, ni, _ = jax.lax.while_loop(
        cond, body, (rv[...], ri[...], m0))
    rv[...] = nv
    ri[...] = ni

    @pl.when(i == nb - 1)
    def _out():
        vals_ref[...] = nv
        idx_ref[...] = ni


def _retrieve(q, p, blk):
    qn, d = q.shape
    k_total = p.shape[0]
    nb = pl.cdiv(k_total, blk)
    vals, idx = pl.pallas_call(
        functools.partial(_topk_kernel, k_total=k_total, blk=blk),
        grid=(nb,),
        in_specs=[
            pl.BlockSpec((qn, d), lambda i: (0, 0)),
            pl.BlockSpec((blk, d), lambda i: (i, 0)),
        ],
        out_specs=[
            pl.BlockSpec((qn, PAD), lambda i: (0, 0)),
            pl.BlockSpec((qn, PAD), lambda i: (0, 0)),
        ],
        out_shape=[
            jax.ShapeDtypeStruct((qn, PAD), jnp.float32),
            jax.ShapeDtypeStruct((qn, PAD), jnp.int32),
        ],
        scratch_shapes=[
            pltpu.VMEM((qn, PAD), jnp.float32),
            pltpu.VMEM((qn, PAD), jnp.int32),
            pltpu.VMEM((qn, blk), jnp.float32),
        ],
        compiler_params=pltpu.CompilerParams(
            dimension_semantics=("arbitrary",),
        ),
    )(q, p)
    return vals[:, :TOPK], idx[:, :TOPK]


def kernel(question_embeddings, passage_embeddings, topk):
    del topk
    logits, retrieved_indices = _retrieve(
        question_embeddings, passage_embeddings, BLK)
    return logits, retrieved_indices, question_embeddings
